# baseline (speedup 1.0000x reference)
import functools

import jax
import jax.numpy as jnp
from jax import lax
from jax.experimental import pallas as pl
from jax.experimental.pallas import tpu as pltpu
from jax.experimental.pallas import tpu_sc as plsc

_LANES = 16
_CHUNK = 128


@functools.cache
def _build(B, D, NC, NS):
    NW = NC * NS
    b_per_w = B // NW
    n_chunks = b_per_w // _CHUNK
    mesh = plsc.VectorSubcoreMesh(core_axis_name="c", subcore_axis_name="s")

    @functools.partial(
        pl.kernel,
        mesh=mesh,
        out_type=(
            jax.ShapeDtypeStruct((B,), jnp.float32),
            jax.ShapeDtypeStruct((B, D), jnp.float32),
            jax.ShapeDtypeStruct((B, D), jnp.float32),
        ),
        scratch_types=[
            pltpu.VMEM((n_chunks, _CHUNK), jnp.int32),
            pltpu.VMEM((n_chunks, _CHUNK), jnp.int32),
            pltpu.VMEM((b_per_w, D), jnp.float32),
            pltpu.VMEM((b_per_w, D), jnp.float32),
            pltpu.VMEM((b_per_w,), jnp.float32),
            pltpu.SemaphoreType.DMA,
            pltpu.SemaphoreType.DMA,
        ],
        compiler_params=pltpu.CompilerParams(
            needs_layout_passes=False, use_tc_tiling_on_sc=False),
    )
    def run(user_h, item_h, gu_h, gi_h, xui_h, gu_out_h, gi_out_h,
            uidx_v, iidx_v, urows_v, irows_v, xui_v, gsem, osem):
        wid = lax.axis_index("s") * NC + lax.axis_index("c")
        base = wid * b_per_w

        pltpu.sync_copy(user_h.at[wid], uidx_v)
        pltpu.sync_copy(item_h.at[wid], iidx_v)

        gathers = []
        for j in range(n_chunks):
            dst = pl.ds(j * _CHUNK, _CHUNK)
            gathers.append(
                pltpu.async_copy(gu_h.at[uidx_v.at[j]], urows_v.at[dst], gsem))
            gathers.append(
                pltpu.async_copy(gi_h.at[iidx_v.at[j]], irows_v.at[dst], gsem))
        for c in gathers:
            c.wait()

        wu = pltpu.async_copy(urows_v, gu_out_h.at[pl.ds(base, b_per_w)], osem)
        wi = pltpu.async_copy(irows_v, gi_out_h.at[pl.ds(base, b_per_w)], osem)

        lane = lax.iota(jnp.int32, _LANES)

        def group(g, carry):
            vec = jnp.zeros((_LANES,), jnp.float32)
            for l in range(_LANES):
                r = g * _LANES + l
                acc = jnp.zeros((_LANES,), jnp.float32)
                for c in range(0, D, _LANES):
                    acc = acc + (urows_v[r, pl.ds(c, _LANES)]
                                 * irows_v[r, pl.ds(c, _LANES)])
                vec = jnp.where(lane == l, jnp.sum(acc), vec)
            xui_v[pl.ds(g * _LANES, _LANES)] = vec
            return carry

        lax.fori_loop(0, b_per_w // _LANES, group, 0)

        pltpu.sync_copy(xui_v, xui_h.at[pl.ds(base, b_per_w)])
        wu.wait()
        wi.wait()

    return run


def kernel(user, item, Gu, Gi):
    B = user.shape[0]
    D = Gu.shape[1]
    info = plsc.get_sparse_core_info()
    run = _build(B, D, info.num_cores, info.num_subcores)
    NW = info.num_cores * info.num_subcores
    user3 = user.reshape(NW, -1, _CHUNK)
    item3 = item.reshape(NW, -1, _CHUNK)
    xui, gamma_u, gamma_i = run(user3, item3, Gu, Gi)
    return (xui, gamma_u, gamma_i)
